# baseline (speedup 1.0000x reference)
import dataclasses

import jax
import jax.numpy as jnp
from jax.experimental import pallas as pl
from jax.experimental.pallas import tpu as pltpu
from jax.experimental.pallas import tpu_sc as plsc

N_NODES = 100000
D = 128
E = 1024
RT = 10000
NBT = N_NODES // RT
WC = 128
CH = 128
NSUB = 32
SEG_PER = E // NSUB
LANES = 16
EPAD = E + WC


def _mm_body(ev_ref, x_ref, wpre_ref, bpre_ref, o_ref, cnt_ref, cnt_scr):
    i = pl.program_id(0)

    @pl.when(i == 0)
    def _init():
        cnt_scr[...] = jnp.zeros_like(cnt_scr)

    xb = x_ref[0].astype(jnp.bfloat16)
    wb = wpre_ref[...].astype(jnp.bfloat16)
    pre = jax.lax.dot_general(xb, wb, (((1,), (0,)), ((), ())),
                              preferred_element_type=jnp.float32)
    o_ref[0] = jnp.maximum(pre + bpre_ref[...], 0.0)

    first = ev_ref[0, 0, 0]
    last = ev_ref[0, 0, RT - 1]
    base = first - first % 8
    n_win = (last - base) // WC + 1
    ev16 = ev_ref[0, 0, :].astype(jnp.int16)
    iota = jax.lax.broadcasted_iota(jnp.int16, (WC, RT), 0)

    def _one_window(w, _):
        ws = base + w * WC
        ev_rel = ev16 - ws.astype(jnp.int16)
        oh = iota == jnp.broadcast_to(ev_rel[None, :], (WC, RT))
        cnt_scr[pl.ds(ws, WC), :] += jnp.sum(
            oh.astype(jnp.float32), axis=1, keepdims=True)
        return 0

    jax.lax.fori_loop(0, n_win, _one_window, 0)

    @pl.when(i == NBT - 1)
    def _fin():
        cnt_ref[...] = cnt_scr[:E, :]


def _pre_nn(x, ev3, W_pre, b_pre):
    return pl.pallas_call(
        _mm_body,
        grid=(NBT,),
        in_specs=[
            pl.BlockSpec((1, 1, RT), lambda i: (i, 0, 0)),
            pl.BlockSpec((1, RT, D), lambda i: (i, 0, 0)),
            pl.BlockSpec((D, D), lambda i: (0, 0)),
            pl.BlockSpec((1, D), lambda i: (0, 0)),
        ],
        out_specs=[
            pl.BlockSpec((1, RT, D), lambda i: (i, 0, 0)),
            pl.BlockSpec((E, 1), lambda i: (0, 0)),
        ],
        out_shape=[
            jax.ShapeDtypeStruct((NBT + 1, RT, D), jnp.float32),
            jax.ShapeDtypeStruct((E, 1), jnp.float32),
        ],
        scratch_shapes=[pltpu.VMEM((EPAD, 1), jnp.float32)],
    )(ev3, x.reshape(NBT, RT, D), W_pre, b_pre.reshape(1, D))


def _sc_scatter(ftx, ev2, bounds):
    vector_mesh = plsc.VectorSubcoreMesh(
        core_axis_name="core", subcore_axis_name="subcore")

    cp = pltpu.CompilerParams()
    if "needs_layout_passes" in pltpu.CompilerParams.__dataclass_fields__:
        cp = dataclasses.replace(cp, needs_layout_passes=False)

    @pl.kernel(
        out_type=jax.ShapeDtypeStruct((E, D), jnp.float32),
        mesh=vector_mesh,
        compiler_params=cp,
        scratch_types=[
            pltpu.VMEM((CH, D), jnp.float32),
            pltpu.VMEM((1, CH), jnp.int32),
            pltpu.VMEM((1, CH), jnp.int32),
            pltpu.VMEM((SEG_PER, D), jnp.float32),
            pltpu.VMEM((1, 4 * LANES), jnp.int32),
            pltpu.VMEM_SHARED((E + NSUB, D), jnp.float32),
            pltpu.SemaphoreType.DMA,
            pltpu.SemaphoreType.DMA,
        ],
    )
    def kern(ftx_hbm, ev_hbm, bounds_hbm, gsum_hbm,
             buf, idx_buf, ev_buf, zslab, bnd, acc, sem0, sem1):
        cid = jax.lax.axis_index("core")
        sid = jax.lax.axis_index("subcore")
        k = cid * 16 + sid

        cp_b = pltpu.make_async_copy(bounds_hbm, bnd, sem0)
        cp_b.start()

        zero = jnp.zeros((LANES,), jnp.float32)

        @pl.loop(0, SEG_PER)
        def _z(r):
            @pl.loop(0, D // LANES)
            def _zc(c):
                zslab[r, pl.ds(c * LANES, LANES)] = zero

        cp_b.wait()
        iota = jax.lax.iota(jnp.int32, LANES)
        kl = jax.lax.rem(k, 16)
        s0 = bnd[0, pl.ds(0, LANES)]
        s1 = bnd[0, pl.ds(LANES, LANES)]
        e0 = bnd[0, pl.ds(2 * LANES, LANES)]
        e1 = bnd[0, pl.ds(3 * LANES, LANES)]

        def _pick(v):
            return jnp.sum(jnp.where(iota == kl, v, 0))

        start = jnp.where(k < 16, _pick(s0), _pick(s1))
        end = jnp.where(k < 16, _pick(e0), _pick(e1))
        start_al = start - start % CH
        nch = (end - start_al + CH - 1) // CH

        cp_z1 = pltpu.make_async_copy(
            zslab, acc.at[pl.ds(k * SEG_PER, SEG_PER), :], sem0)
        cp_z1.start()
        cp_z1.wait()
        cp_d1 = pltpu.make_async_copy(
            zslab.at[pl.ds(0, 1), :], acc.at[pl.ds(E + k, 1), :], sem0)
        cp_d1.start()
        cp_d1.wait()

        @pl.loop(0, nch)
        def _chunk(c):
            s_c = pl.multiple_of(start_al + c * CH, CH)
            cp_x = pltpu.make_async_copy(
                ftx_hbm.at[pl.ds(s_c, CH), :], buf, sem0)
            cp_e = pltpu.make_async_copy(
                ev_hbm.at[0, pl.ds(s_c, CH)], ev_buf.at[0], sem1)
            cp_x.start()
            cp_e.start()
            cp_x.wait()
            cp_e.wait()

            @pl.loop(0, CH // LANES)
            def _g(g):
                gvec = iota + (s_c + g * LANES)
                evv = ev_buf[0, pl.ds(g * LANES, LANES)]
                valid = (gvec >= start) & (gvec < end)
                idx_buf[0, pl.ds(g * LANES, LANES)] = jnp.where(
                    valid, evv, E + k)

            cp_s = pltpu.make_async_copy(
                buf, acc.at[idx_buf.at[0]], sem0)
            cp_s.start(add=True)
            cp_s.wait()

        cp_out = pltpu.make_async_copy(
            acc.at[pl.ds(k * SEG_PER, SEG_PER), :],
            gsum_hbm.at[pl.ds(k * SEG_PER, SEG_PER), :], sem0)
        cp_out.start()
        cp_out.wait()

    return kern(ftx, ev2, bounds)


def _finish_body(gsum_ref, cnt_ref, wpost_ref, bpost_ref, out_ref):
    gsum = gsum_ref[...]
    gmean = gsum * (1.0 / jnp.maximum(cnt_ref[...], 1.0))
    w1 = wpost_ref[:D, :]
    w2 = wpost_ref[D:, :]
    out_ref[...] = (
        jax.lax.dot_general(gsum, w1, (((1,), (0,)), ((), ())),
                            preferred_element_type=jnp.float32)
        + jax.lax.dot_general(gmean, w2, (((1,), (0,)), ((), ())),
                              preferred_element_type=jnp.float32)
        + bpost_ref[...])


def kernel(x, event, W_pre, b_pre, W_post, b_post):
    ev = event.astype(jnp.int32)
    ftx, cnt = _pre_nn(x, ev.reshape(NBT, 1, RT), W_pre, b_pre)
    seg_edges = jnp.arange(0, E + 1, SEG_PER, dtype=jnp.int32)
    b33 = jnp.searchsorted(ev, seg_edges, side="left").astype(jnp.int32)
    bounds = jnp.concatenate([b33[:NSUB], b33[1:NSUB + 1]]).reshape(1, 64)
    ev_pad = jnp.concatenate([ev, jnp.zeros((RT,), jnp.int32)])
    gsum = _sc_scatter(ftx.reshape((NBT + 1) * RT, D),
                       ev_pad.reshape(1, N_NODES + RT), bounds)
    return pl.pallas_call(
        _finish_body,
        grid=(1,),
        in_specs=[
            pl.BlockSpec((E, D), lambda i: (0, 0)),
            pl.BlockSpec((E, 1), lambda i: (0, 0)),
            pl.BlockSpec((2 * D, D), lambda i: (0, 0)),
            pl.BlockSpec((1, D), lambda i: (0, 0)),
        ],
        out_specs=pl.BlockSpec((E, D), lambda i: (0, 0)),
        out_shape=jax.ShapeDtypeStruct((E, D), jnp.float32),
    )(gsum, cnt, W_post, b_post.reshape(1, D))


# speedup vs baseline: 3.8756x; 3.8756x over previous
import jax
import jax.numpy as jnp
from jax.experimental import pallas as pl
from jax.experimental.pallas import tpu as pltpu

N_NODES = 100000
D = 128
E = 1024
R = 10000
NB = N_NODES // R
W = 128
EPAD = E + W


def _body(ev_ref, x_ref, wpre_ref, bpre_ref, wpost_ref, bpost_ref,
          out_ref, gsum_ref, cnt_ref, ftx_ref):
    i = pl.program_id(0)

    @pl.when(i == 0)
    def _init():
        gsum_ref[...] = jnp.zeros_like(gsum_ref)
        cnt_ref[...] = jnp.zeros_like(cnt_ref)
        ftx_ref[:, D:] = jnp.ones((R, D), jnp.bfloat16)

    xb = x_ref[0].astype(jnp.bfloat16)
    wb = wpre_ref[...].astype(jnp.bfloat16)
    pre = jax.lax.dot_general(xb, wb, (((1,), (0,)), ((), ())),
                              preferred_element_type=jnp.float32)
    ftx_ref[:, :D] = jnp.maximum(pre + bpre_ref[...], 0.0).astype(jnp.bfloat16)

    first = ev_ref[0, 0, 0]
    last = ev_ref[0, 0, R - 1]
    base = first - first % 8
    n_win = (last - base) // W + 1
    ev16 = ev_ref[0, 0, :].astype(jnp.int16)

    def _one_window(w, _):
        ws = base + w * W
        ev_rel = ev16 - ws.astype(jnp.int16)
        iota = jax.lax.broadcasted_iota(jnp.int16, (W, R), 0)
        ohb = (iota == jnp.broadcast_to(ev_rel[None, :], (W, R))
               ).astype(jnp.bfloat16)
        h = R // 4
        parts = [jax.lax.dot_general(ohb[:, j * h:(j + 1) * h],
                                     ftx_ref[j * h:(j + 1) * h, :],
                                     (((1,), (0,)), ((), ())),
                                     preferred_element_type=jnp.float32)
                 for j in range(4)]
        res = (parts[0] + parts[1]) + (parts[2] + parts[3])
        gsum_ref[pl.ds(ws, W), :] += res[:, :D]
        cnt_ref[pl.ds(ws, W), :] += res[:, D:D + 1]
        return 0

    jax.lax.fori_loop(0, n_win, _one_window, 0)

    @pl.when(i == NB - 1)
    def _finish():
        gsum = gsum_ref[:E, :]
        gmean = gsum * (1.0 / jnp.maximum(cnt_ref[:E, :], 1.0))
        w1 = wpost_ref[:D, :]
        w2 = wpost_ref[D:, :]
        out_ref[...] = (
            jax.lax.dot_general(gsum, w1, (((1,), (0,)), ((), ())),
                                preferred_element_type=jnp.float32)
            + jax.lax.dot_general(gmean, w2, (((1,), (0,)), ((), ())),
                                  preferred_element_type=jnp.float32)
            + bpost_ref[...])


def kernel(x, event, W_pre, b_pre, W_post, b_post):
    ev = event.astype(jnp.int32)
    return pl.pallas_call(
        _body,
        grid=(NB,),
        in_specs=[
            pl.BlockSpec((1, 1, R), lambda i: (i, 0, 0)),
            pl.BlockSpec((1, R, D), lambda i: (i, 0, 0)),
            pl.BlockSpec((D, D), lambda i: (0, 0)),
            pl.BlockSpec((1, D), lambda i: (0, 0)),
            pl.BlockSpec((2 * D, D), lambda i: (0, 0)),
            pl.BlockSpec((1, D), lambda i: (0, 0)),
        ],
        out_specs=pl.BlockSpec((E, D), lambda i: (0, 0)),
        out_shape=jax.ShapeDtypeStruct((E, D), jnp.float32),
        scratch_shapes=[
            pltpu.VMEM((EPAD, D), jnp.float32),
            pltpu.VMEM((EPAD, 1), jnp.float32),
            pltpu.VMEM((R, 2 * D), jnp.bfloat16),
        ],
    )(ev.reshape(NB, 1, R), x.reshape(NB, R, D), W_pre,
      b_pre.reshape(1, D), W_post, b_post.reshape(1, D))
